# baseline (speedup 1.0000x reference)
import functools
import numpy as np
import jax
import jax.numpy as jnp
from jax import lax
from jax.experimental import pallas as pl
from jax.experimental.pallas import tpu as pltpu
from jax.experimental.pallas import tpu_sc as plsc

NN = 20000
F32 = jnp.float32
NC = 2
NS = 16
NW = NC * NS
CHUNK = 80
BLK = 1000
NBLK = NN // BLK
BM = 2000
GG = NN // BM

NTS = ("ind", "org", "ext")
ROLES = [("ind", 0), ("ind", 1), ("ind", 2), ("ind", 3),
         ("org", 0), ("org", 1), ("ext", 0), ("ext", 1)]
ROLE_IDX = {r: i for i, r in enumerate(ROLES)}
NROLE = {"ind": 4, "org": 2, "ext": 2}
ETS_C1 = [
    ("ind", "org", 640000, 0, 1),
    ("org", "ind", 640000, 0, 2),
    ("ext", "ind", 320000, 0, 3),
    ("ind", "ext", 320000, 1, 1),
]
ETS_C2 = ETS_C1[:3]


def _emit_block(src, wps, bps, avecs, rec_refs, alpha_refs):
  for i, nt in enumerate(NTS):
    xn = jnp.dot(src[nt], wps[i][:], preferred_element_type=F32) + bps[i][:]
    rec_refs[i][:, :] = xn
    for r in range(NROLE[nt]):
      av = avecs.get((nt, r))
      alpha_refs[ROLE_IDX[(nt, r)]][:, :] = (
          jnp.zeros((BM, 1), F32) if av is None
          else jnp.dot(xn, av[:], preferred_element_type=F32))


def _pre1_body(x_i, x_o, x_e, wp_i, wp_o, wp_e, bp_i, bp_o, bp_e,
               as_txn, ad_txn, as_rt, ad_rt, as_l, ad_l, as_rl, ad_rl,
               *outs):
  src = {"ind": x_i[:], "org": x_o[:], "ext": x_e[:]}
  avecs = {("ind", 0): as_txn, ("ind", 1): as_rl, ("ind", 2): ad_rt,
           ("ind", 3): ad_l, ("org", 0): as_rt, ("org", 1): ad_txn,
           ("ext", 0): as_l, ("ext", 1): ad_rl}
  _emit_block(src, (wp_i, wp_o, wp_e), (bp_i, bp_o, bp_e), avecs,
              outs[0:3], outs[3:11])


def _seg_out(acc_ref, den_ref, k):
  num = acc_ref[0, k] + acc_ref[1, k]
  den = den_ref[0, k] + den_ref[1, k]
  return jnp.maximum(num / (den + 1e-16), 0.0)


def _score_part(o, kW, kb, q):
  t = jnp.tanh(jnp.dot(o, kW, preferred_element_type=F32) + kb)
  return jnp.sum(t * q)


def _attn2(sm):
  m1 = sm[0] / NN
  m2 = sm[1] / NN
  mm = jnp.maximum(m1, m2)
  e1 = jnp.exp(m1 - mm)
  e2 = jnp.exp(m2 - mm)
  return e1 / (e1 + e2), e2 / (e1 + e2)


def _mid_body(acc, dend, kW, kb, q,
              l1iw, l1ib, l1ow, l1ob, l1ew, l1eb,
              wp_i, wp_o, wp_e, bp_i, bp_o, bp_e,
              as_txn, ad_txn, as_rt, ad_rt, as_l, ad_l,
              *outs_and_scratch):
  outs = outs_and_scratch[:-1]
  sm = outs_and_scratch[-1]
  p = pl.program_id(0)
  g = pl.program_id(1)
  o_rt = _seg_out(acc, dend, 1)
  o_l = _seg_out(acc, dend, 2)

  @pl.when(jnp.logical_and(p == 0, g == 0))
  def _init():
    sm[0] = 0.0
    sm[1] = 0.0

  @pl.when(p == 0)
  def _accum():
    sm[0] = sm[0] + _score_part(o_rt, kW[:], kb[:], q[:])
    sm[1] = sm[1] + _score_part(o_l, kW[:], kb[:], q[:])

  @pl.when(p == 1)
  def _emit():
    a1, a2 = _attn2(sm)
    res = {"org": _seg_out(acc, dend, 0),
           "ind": a1 * o_rt + a2 * o_l,
           "ext": _seg_out(acc, dend, 3)}
    lin = {"ind": (l1iw, l1ib), "org": (l1ow, l1ob), "ext": (l1ew, l1eb)}
    src = {nt: jax.nn.sigmoid(
        jnp.dot(res[nt], lin[nt][0][:], preferred_element_type=F32)
        + lin[nt][1][:]) for nt in NTS}
    avecs = {("ind", 0): as_txn, ("ind", 2): ad_rt, ("ind", 3): ad_l,
             ("org", 0): as_rt, ("org", 1): ad_txn, ("ext", 0): as_l}
    _emit_block(src, (wp_i, wp_o, wp_e), (bp_i, bp_o, bp_e), avecs,
                outs[0:3], outs[3:11])


def _fin_body(acc, dend, kW, kb, q, l2iw, l2ib, l2ow, l2ob,
              zi_ref, zo_ref, sm):
  p = pl.program_id(0)
  g = pl.program_id(1)
  o_rt = _seg_out(acc, dend, 1)
  o_l = _seg_out(acc, dend, 2)

  @pl.when(jnp.logical_and(p == 0, g == 0))
  def _init():
    sm[0] = 0.0
    sm[1] = 0.0

  @pl.when(p == 0)
  def _accum():
    sm[0] = sm[0] + _score_part(o_rt, kW[:], kb[:], q[:])
    sm[1] = sm[1] + _score_part(o_l, kW[:], kb[:], q[:])

  @pl.when(p == 1)
  def _emit():
    a1, a2 = _attn2(sm)
    res_ind = a1 * o_rt + a2 * o_l
    zi_ref[:, :] = jax.nn.sigmoid(
        jnp.dot(res_ind, l2iw[:], preferred_element_type=F32) + l2ib[:])
    zo_ref[:, :] = jax.nn.sigmoid(
        jnp.dot(_seg_out(acc, dend, 0), l2ow[:], preferred_element_type=F32)
        + l2ob[:])


MAXCH = 250


def _make_edge_kernel(ets):
  ET = len(ets)
  mesh = plsc.VectorSubcoreMesh(core_axis_name="c", subcore_axis_name="s")
  scratch = [
      pltpu.VMEM((NN,), F32),
      pltpu.VMEM((NN,), F32),
      pltpu.VMEM((2, MAXCH, CHUNK), jnp.int32),
      pltpu.VMEM_SHARED((NN, 8), F32),
      pltpu.VMEM_SHARED((NN,), F32),
  ] + [
      pltpu.VMEM((CHUNK, 8), F32), pltpu.VMEM((CHUNK,), F32),
      pltpu.SemaphoreType.DMA, pltpu.SemaphoreType.DMA,
  ] * 2

  def body(*refs):
    eis = [refs[3 * k] for k in range(ET)]
    asrcs = [refs[3 * k + 1] for k in range(ET)]
    adsts = [refs[3 * k + 2] for k in range(ET)]
    rec = {"ind": refs[3 * ET], "org": refs[3 * ET + 1],
           "ext": refs[3 * ET + 2]}
    zeros8 = refs[3 * ET + 3]
    zerosd = refs[3 * ET + 4]
    out = refs[3 * ET + 5]
    out_den = refs[3 * ET + 6]
    (asrc_t, adst_t, idx_all, acc, accd) = (
        refs[3 * ET + 7: 3 * ET + 12])
    bufs = [refs[3 * ET + 12: 3 * ET + 16], refs[3 * ET + 16: 3 * ET + 20]]

    c = lax.axis_index("c")
    s = lax.axis_index("s")
    w = c * NS + s

    def table_max(tbl):
      def mbody(r, m):
        return jnp.maximum(m, tbl[pl.ds(r * 16, 16)])
      m = lax.fori_loop(0, NN // 16, mbody, jnp.full((16,), -3.4e38, F32))
      return jnp.max(m)

    for k, (snt, dnt, ne, rs, rd) in enumerate(ets):
      for t in range(-(-NBLK // NS)):
        b = s + NS * t
        @pl.when(b < NBLK)
        def _zero():
          pltpu.sync_copy(zeros8, acc.at[pl.ds(b * BLK, BLK)])
          pltpu.sync_copy(zerosd, accd.at[pl.ds(b * BLK, BLK)])
      pltpu.sync_copy(asrcs[k], asrc_t)
      pltpu.sync_copy(adsts[k], adst_t)
      nch = ne // NW // CHUNK
      pltpu.sync_copy(eis[k].at[:, pl.ds(w * nch, nch)],
                      idx_all.at[:, pl.ds(0, nch)])
      sv = jnp.full(
          (16,),
          jnp.maximum(table_max(asrc_t) + table_max(adst_t), 0.0), F32)
      rec_hbm = rec[snt]
      plsc.subcore_barrier()

      def gather(ci, buf):
        rows, exb, gsem, ssem = buf
        pltpu.async_copy(rec_hbm.at[idx_all.at[0, ci]], rows, gsem)

      def gather_wait(ci, buf):
        rows, exb, gsem, ssem = buf
        pltpu.make_async_copy(rec_hbm.at[idx_all.at[0, ci]], rows,
                              gsem).wait()

      def scatter(ci, buf):
        rows, exb, gsem, ssem = buf
        pltpu.async_copy(rows, acc.at[idx_all.at[1, ci]], ssem, add=True)
        pltpu.async_copy(exb, accd.at[idx_all.at[1, ci]], ssem, add=True)

      def scatter_wait(ci, buf):
        rows, exb, gsem, ssem = buf
        pltpu.make_async_copy(rows, acc.at[idx_all.at[1, ci]], ssem).wait()
        pltpu.make_async_copy(exb, accd.at[idx_all.at[1, ci]], ssem).wait()

      def alpha_pass(ci, buf, sv=sv):
        rows, exb, gsem, ssem = buf
        for j in range(CHUNK // 16):
          sl = pl.ds(j * 16, 16)
          a = (plsc.load_gather(asrc_t, [idx_all[0, ci, sl]])
               + plsc.load_gather(adst_t, [idx_all[1, ci, sl]]))
          a = jnp.where(a >= 0.0, a, a * 0.2)
          exb[sl] = jnp.exp(a - sv)

      lane = lax.broadcasted_iota(jnp.int32, (16,), 0)
      ge8 = (lane >= 8).astype(jnp.int32)
      cidx = lane & 7

      def escale(buf):
        rows, exb, gsem, ssem = buf

        @plsc.parallel_loop(0, CHUNK // 2, 1, unroll=8)
        def _loop(e):
          eidx = jnp.full((16,), 2 * e, jnp.int32) + ge8
          bex = plsc.load_gather(exb, [eidx])
          v = plsc.load_gather(rows, [eidx, cidx])
          plsc.store_scatter(rows, [eidx, cidx], v * bex)

      gather(0, bufs[0])

      def pair_body(i, carry):
        c0 = 2 * i
        alpha_pass(c0, bufs[0])
        gather_wait(c0, bufs[0])

        @pl.when(i > 0)
        def _drain_b():
          scatter_wait(c0 - 1, bufs[1])
        gather(c0 + 1, bufs[1])
        escale(bufs[0])
        scatter(c0, bufs[0])
        alpha_pass(c0 + 1, bufs[1])
        gather_wait(c0 + 1, bufs[1])
        scatter_wait(c0, bufs[0])

        @pl.when(c0 + 2 < nch)
        def _pref():
          gather(c0 + 2, bufs[0])
        escale(bufs[1])
        scatter(c0 + 1, bufs[1])
        return carry
      lax.fori_loop(0, nch // 2, pair_body, 0)
      if nch % 2:
        ci = nch - 1
        alpha_pass(ci, bufs[0])
        gather_wait(ci, bufs[0])
        scatter_wait(ci - 1, bufs[1])
        escale(bufs[0])
        scatter(ci, bufs[0])
        scatter_wait(ci, bufs[0])
      else:
        scatter_wait(nch - 1, bufs[1])

      plsc.subcore_barrier()
      for t in range(-(-NBLK // NS)):
        b = s + NS * t
        @pl.when(b < NBLK)
        def _readout():
          pltpu.sync_copy(acc.at[pl.ds(b * BLK, BLK)],
                          out.at[c, k, pl.ds(b * BLK, BLK)])
          pltpu.sync_copy(accd.at[pl.ds(b * BLK, BLK)],
                          out_den.at[c, k, pl.ds(b * BLK, BLK)])

  return pl.kernel(
      body,
      out_type=(jax.ShapeDtypeStruct((NC, ET, NN, 8), F32),
                jax.ShapeDtypeStruct((NC, ET, NN), F32)),
      mesh=mesh,
      scratch_types=scratch,
      compiler_params=pltpu.CompilerParams(needs_layout_passes=False,
                                           use_tc_tiling_on_sc=False),
  )


_EDGE_K1 = _make_edge_kernel(ETS_C1)
_EDGE_K2 = _make_edge_kernel(ETS_C2)

_full = lambda shp: pl.BlockSpec(shp, lambda *_: tuple(0 for _ in shp))
_rows = lambda shp: pl.BlockSpec(shp, lambda *gi: (gi[-1],) + (0,) * (len(shp) - 1))

_REC_SDS = jax.ShapeDtypeStruct((NN, 8), F32)
_AL_SDS = jax.ShapeDtypeStruct((NN, 1), F32)
_TBL_OUT = ([_REC_SDS] * 3) + ([_AL_SDS] * 8)
_TBL_OUT_SPECS = ([_rows((BM, 8))] * 3) + ([_rows((BM, 1))] * 8)


def kernel(x_ind, x_org, x_ext, edge_index_txn, edge_index_rev_txn,
           edge_index_link, edge_index_rev_link, edge_attr_txn,
           c1_Wp_ind, c1_Wp_org, c1_Wp_ext, c1_bp_ind, c1_bp_org, c1_bp_ext,
           c1_asrc_ind__txn__org, c1_adst_ind__txn__org,
           c1_asrc_org__rev_txn__ind, c1_adst_org__rev_txn__ind,
           c1_asrc_ext__link__ind, c1_adst_ext__link__ind,
           c1_asrc_ind__rev_link__ext, c1_adst_ind__rev_link__ext,
           c1_kW, c1_kb, c1_q,
           c2_Wp_ind, c2_Wp_org, c2_Wp_ext, c2_bp_ind, c2_bp_org, c2_bp_ext,
           c2_asrc_ind__txn__org, c2_adst_ind__txn__org,
           c2_asrc_org__rev_txn__ind, c2_adst_org__rev_txn__ind,
           c2_asrc_ext__link__ind, c2_adst_ext__link__ind,
           c2_asrc_ind__rev_link__ext, c2_adst_ind__rev_link__ext,
           c2_kW, c2_kb, c2_q,
           lin_ind_1_W, lin_ind_1_b, lin_org_1_W, lin_org_1_b,
           lin_ext_1_W, lin_ext_1_b,
           lin_ind_2_W, lin_ind_2_b, lin_org_2_W, lin_org_2_b):
  col = lambda a: a.reshape(8, 1)

  t1 = pl.pallas_call(
      _pre1_body,
      grid=(GG,),
      in_specs=[_rows((BM, 128))] * 3 + [_full((128, 8))] * 3
      + [_full((8,))] * 3 + [_full((8, 1))] * 8,
      out_specs=_TBL_OUT_SPECS,
      out_shape=_TBL_OUT,
  )(x_ind, x_org, x_ext, c1_Wp_ind, c1_Wp_org, c1_Wp_ext,
    c1_bp_ind, c1_bp_org, c1_bp_ext,
    col(c1_asrc_ind__txn__org), col(c1_adst_ind__txn__org),
    col(c1_asrc_org__rev_txn__ind), col(c1_adst_org__rev_txn__ind),
    col(c1_asrc_ext__link__ind), col(c1_adst_ext__link__ind),
    col(c1_asrc_ind__rev_link__ext), col(c1_adst_ind__rev_link__ext))
  recs1, alphas1 = t1[0:3], t1[3:11]

  eis = (edge_index_txn, edge_index_rev_txn, edge_index_link,
         edge_index_rev_link)

  zeros8 = jnp.zeros((BLK, 8), F32)
  zerosd = jnp.zeros((BLK,), F32)

  def edge_inputs(ets, recs, alphas):
    args = []
    for ei, (snt, dnt, ne, rs, rd) in zip(eis, ets):
      args += [ei.reshape(2, ne // CHUNK, CHUNK),
               alphas[ROLE_IDX[(snt, rs)]].reshape(NN),
               alphas[ROLE_IDX[(dnt, rd)]].reshape(NN)]
    return args + list(recs) + [zeros8, zerosd]

  acc1, den1 = _EDGE_K1(*edge_inputs(ETS_C1, recs1, alphas1))

  t2 = pl.pallas_call(
      _mid_body,
      grid=(2, GG),
      in_specs=[pl.BlockSpec((2, 4, BM, 8), lambda p, g: (0, 0, g, 0)),
                pl.BlockSpec((2, 4, BM, 1), lambda p, g: (0, 0, g, 0))]
      + [_full((8, 8)), _full((8,)), _full((8,))]
      + [_full((8, 128)), _full((128,))] * 3
      + [_full((128, 8))] * 3 + [_full((8,))] * 3 + [_full((8, 1))] * 6,
      out_specs=_TBL_OUT_SPECS,
      out_shape=_TBL_OUT,
      scratch_shapes=[pltpu.SMEM((2,), F32)],
  )(acc1, den1.reshape(NC, 4, NN, 1), c1_kW, c1_kb, c1_q,
    lin_ind_1_W, lin_ind_1_b, lin_org_1_W, lin_org_1_b,
    lin_ext_1_W, lin_ext_1_b,
    c2_Wp_ind, c2_Wp_org, c2_Wp_ext, c2_bp_ind, c2_bp_org, c2_bp_ext,
    col(c2_asrc_ind__txn__org), col(c2_adst_ind__txn__org),
    col(c2_asrc_org__rev_txn__ind), col(c2_adst_org__rev_txn__ind),
    col(c2_asrc_ext__link__ind), col(c2_adst_ext__link__ind))
  recs2, alphas2 = t2[0:3], t2[3:11]

  acc2, den2 = _EDGE_K2(*edge_inputs(ETS_C2, recs2, alphas2))

  zi, zo = pl.pallas_call(
      _fin_body,
      grid=(2, GG),
      in_specs=[pl.BlockSpec((2, 3, BM, 8), lambda p, g: (0, 0, g, 0)),
                pl.BlockSpec((2, 3, BM, 1), lambda p, g: (0, 0, g, 0))]
      + [_full((8, 8)), _full((8,)), _full((8,))]
      + [_full((8, 1)), _full((1,))] * 2,
      out_specs=[_rows((BM, 1))] * 2,
      out_shape=[jax.ShapeDtypeStruct((NN, 1), F32),
                 jax.ShapeDtypeStruct((NN, 1), F32)],
      scratch_shapes=[pltpu.SMEM((2,), F32)],
  )(acc2, den2.reshape(NC, 3, NN, 1), c2_kW, c2_kb, c2_q,
    lin_ind_2_W, lin_ind_2_b, lin_org_2_W, lin_org_2_b)
  return (zi[:, 0], zo[:, 0])


# speedup vs baseline: 1.7452x; 1.7452x over previous
import functools
import numpy as np
import jax
import jax.numpy as jnp
from jax import lax
from jax.experimental import pallas as pl
from jax.experimental.pallas import tpu as pltpu
from jax.experimental.pallas import tpu_sc as plsc

NN = 20000
F32 = jnp.float32
NC = 2
NS = 16
NW = NC * NS
CHUNK = 80
BLK = 1000
NBLK = NN // BLK
BM = 2000
GG = NN // BM

NTS = ("ind", "org", "ext")
ROLES = [("ind", 0), ("ind", 1), ("ind", 2), ("ind", 3),
         ("org", 0), ("org", 1), ("ext", 0), ("ext", 1)]
ROLE_IDX = {r: i for i, r in enumerate(ROLES)}
NROLE = {"ind": 4, "org": 2, "ext": 2}
ETS_C1 = [
    ("ind", "org", 640000, 0, 1),
    ("org", "ind", 640000, 0, 2),
    ("ext", "ind", 320000, 0, 3),
    ("ind", "ext", 320000, 1, 1),
]
ETS_C2 = ETS_C1[:3]


def _emit_block(src, wps, bps, avecs, rec_refs, alpha_refs):
  for i, nt in enumerate(NTS):
    xn = jnp.dot(src[nt], wps[i][:], preferred_element_type=F32) + bps[i][:]
    rec_refs[i][:, :] = jnp.concatenate(
        [xn, jnp.ones((BM, 1), F32), jnp.zeros((BM, 7), F32)], axis=1)
    for r in range(NROLE[nt]):
      av = avecs.get((nt, r))
      alpha_refs[ROLE_IDX[(nt, r)]][:, :] = (
          jnp.zeros((BM, 1), F32) if av is None
          else jnp.dot(xn, av[:], preferred_element_type=F32))


def _pre1_body(x_i, x_o, x_e, wp_i, wp_o, wp_e, bp_i, bp_o, bp_e,
               as_txn, ad_txn, as_rt, ad_rt, as_l, ad_l, as_rl, ad_rl,
               *outs):
  src = {"ind": x_i[:], "org": x_o[:], "ext": x_e[:]}
  avecs = {("ind", 0): as_txn, ("ind", 1): as_rl, ("ind", 2): ad_rt,
           ("ind", 3): ad_l, ("org", 0): as_rt, ("org", 1): ad_txn,
           ("ext", 0): as_l, ("ext", 1): ad_rl}
  _emit_block(src, (wp_i, wp_o, wp_e), (bp_i, bp_o, bp_e), avecs,
              outs[0:3], outs[3:11])


def _seg_out(acc_ref, k):
  a0 = acc_ref[0, k]
  a1 = acc_ref[1, k]
  num = a0[:, 0:8] + a1[:, 0:8]
  den = a0[:, 8:9] + a1[:, 8:9]
  return jnp.maximum(num / (den + 1e-16), 0.0)


def _score_part(o, kW, kb, q):
  t = jnp.tanh(jnp.dot(o, kW, preferred_element_type=F32) + kb)
  return jnp.sum(t * q)


def _attn2(sm):
  m1 = sm[0] / NN
  m2 = sm[1] / NN
  mm = jnp.maximum(m1, m2)
  e1 = jnp.exp(m1 - mm)
  e2 = jnp.exp(m2 - mm)
  return e1 / (e1 + e2), e2 / (e1 + e2)


def _mid_body(acc, kW, kb, q,
              l1iw, l1ib, l1ow, l1ob, l1ew, l1eb,
              wp_i, wp_o, wp_e, bp_i, bp_o, bp_e,
              as_txn, ad_txn, as_rt, ad_rt, as_l, ad_l,
              *outs_and_scratch):
  outs = outs_and_scratch[:-1]
  sm = outs_and_scratch[-1]
  p = pl.program_id(0)
  g = pl.program_id(1)
  o_rt = _seg_out(acc, 1)
  o_l = _seg_out(acc, 2)

  @pl.when(jnp.logical_and(p == 0, g == 0))
  def _init():
    sm[0] = 0.0
    sm[1] = 0.0

  @pl.when(p == 0)
  def _accum():
    sm[0] = sm[0] + _score_part(o_rt, kW[:], kb[:], q[:])
    sm[1] = sm[1] + _score_part(o_l, kW[:], kb[:], q[:])

  @pl.when(p == 1)
  def _emit():
    a1, a2 = _attn2(sm)
    res = {"org": _seg_out(acc, 0),
           "ind": a1 * o_rt + a2 * o_l,
           "ext": _seg_out(acc, 3)}
    lin = {"ind": (l1iw, l1ib), "org": (l1ow, l1ob), "ext": (l1ew, l1eb)}
    src = {nt: jax.nn.sigmoid(
        jnp.dot(res[nt], lin[nt][0][:], preferred_element_type=F32)
        + lin[nt][1][:]) for nt in NTS}
    avecs = {("ind", 0): as_txn, ("ind", 2): ad_rt, ("ind", 3): ad_l,
             ("org", 0): as_rt, ("org", 1): ad_txn, ("ext", 0): as_l}
    _emit_block(src, (wp_i, wp_o, wp_e), (bp_i, bp_o, bp_e), avecs,
                outs[0:3], outs[3:11])


def _fin_body(acc, kW, kb, q, l2iw, l2ib, l2ow, l2ob, zi_ref, zo_ref, sm):
  p = pl.program_id(0)
  g = pl.program_id(1)
  o_rt = _seg_out(acc, 1)
  o_l = _seg_out(acc, 2)

  @pl.when(jnp.logical_and(p == 0, g == 0))
  def _init():
    sm[0] = 0.0
    sm[1] = 0.0

  @pl.when(p == 0)
  def _accum():
    sm[0] = sm[0] + _score_part(o_rt, kW[:], kb[:], q[:])
    sm[1] = sm[1] + _score_part(o_l, kW[:], kb[:], q[:])

  @pl.when(p == 1)
  def _emit():
    a1, a2 = _attn2(sm)
    res_ind = a1 * o_rt + a2 * o_l
    zi_ref[:, :] = jax.nn.sigmoid(
        jnp.dot(res_ind, l2iw[:], preferred_element_type=F32) + l2ib[:])
    zo_ref[:, :] = jax.nn.sigmoid(
        jnp.dot(_seg_out(acc, 0), l2ow[:], preferred_element_type=F32)
        + l2ob[:])


MAXCH = 250


def _make_edge_kernel(ets):
  ET = len(ets)
  mesh = plsc.VectorSubcoreMesh(core_axis_name="c", subcore_axis_name="s")
  NB = 4
  scratch = [
      pltpu.VMEM((NN,), F32),
      pltpu.VMEM((NN,), F32),
      pltpu.VMEM((2, MAXCH, CHUNK), jnp.int32),
      pltpu.VMEM_SHARED((NN, 16), F32),
  ] + [
      pltpu.VMEM((CHUNK, 16), F32), pltpu.VMEM((CHUNK,), F32),
      pltpu.SemaphoreType.DMA, pltpu.SemaphoreType.DMA,
  ] * NB

  def body(*refs):
    eis = [refs[3 * k] for k in range(ET)]
    asrcs = [refs[3 * k + 1] for k in range(ET)]
    adsts = [refs[3 * k + 2] for k in range(ET)]
    rec = {"ind": refs[3 * ET], "org": refs[3 * ET + 1],
           "ext": refs[3 * ET + 2]}
    zeros16 = refs[3 * ET + 3]
    out = refs[3 * ET + 4]
    (asrc_t, adst_t, idx_all, acc) = refs[3 * ET + 5: 3 * ET + 9]
    base = 3 * ET + 9
    bufs = [refs[base + 4 * i: base + 4 * (i + 1)] for i in range(NB)]

    c = lax.axis_index("c")
    s = lax.axis_index("s")
    w = c * NS + s

    def table_max(tbl):
      def mbody(r, m):
        return jnp.maximum(m, tbl[pl.ds(r * 16, 16)])
      m = lax.fori_loop(0, NN // 16, mbody, jnp.full((16,), -3.4e38, F32))
      return jnp.max(m)

    for k, (snt, dnt, ne, rs, rd) in enumerate(ets):
      for t in range(-(-NBLK // NS)):
        b = s + NS * t
        @pl.when(b < NBLK)
        def _zero():
          pltpu.sync_copy(zeros16, acc.at[pl.ds(b * BLK, BLK)])
      pltpu.sync_copy(asrcs[k], asrc_t)
      pltpu.sync_copy(adsts[k], adst_t)
      nch = ne // NW // CHUNK
      pltpu.sync_copy(eis[k].at[:, pl.ds(w * nch, nch)],
                      idx_all.at[:, pl.ds(0, nch)])
      sv = jnp.full(
          (16,),
          jnp.maximum(table_max(asrc_t) + table_max(adst_t), 0.0), F32)
      rec_hbm = rec[snt]
      plsc.subcore_barrier()

      def gather(ci, buf):
        rows, exb, gsem, ssem = buf
        pltpu.async_copy(rec_hbm.at[idx_all.at[0, ci]], rows, gsem)

      def gather_wait(ci, buf):
        rows, exb, gsem, ssem = buf
        pltpu.make_async_copy(rec_hbm.at[idx_all.at[0, ci]], rows,
                              gsem).wait()

      def scatter(ci, buf):
        rows, exb, gsem, ssem = buf
        pltpu.async_copy(rows, acc.at[idx_all.at[1, ci]], ssem, add=True)

      def scatter_wait(ci, buf):
        rows, exb, gsem, ssem = buf
        pltpu.make_async_copy(rows, acc.at[idx_all.at[1, ci]], ssem).wait()

      def alpha_pass(ci, buf, sv=sv):
        rows, exb, gsem, ssem = buf
        for j in range(CHUNK // 16):
          sl = pl.ds(j * 16, 16)
          a = (plsc.load_gather(asrc_t, [idx_all[0, ci, sl]])
               + plsc.load_gather(adst_t, [idx_all[1, ci, sl]]))
          a = jnp.where(a >= 0.0, a, a * 0.2)
          exb[sl] = jnp.exp(a - sv)

      def escale(buf):
        rows, exb, gsem, ssem = buf

        @plsc.parallel_loop(0, CHUNK, 1, unroll=8)
        def _loop(e):
          bex = plsc.load_gather(exb, [jnp.full((16,), e, jnp.int32)])
          rows[e, :] = rows[e, :] * bex

      for u in range(NB - 1):
        gather(u, bufs[u])

      def step(ci, u, first):
        alpha_pass(ci, bufs[u])
        gather_wait(ci, bufs[u])
        pv = (u + NB - 1) % NB
        if first:
          @pl.when(ci >= 1)
          def _drain():
            scatter_wait(ci - 1, bufs[pv])
        else:
          scatter_wait(ci - 1, bufs[pv])

        @pl.when(ci + NB - 1 < nch)
        def _pref():
          gather(ci + NB - 1, bufs[pv])
        escale(bufs[u])
        scatter(ci, bufs[u])

      def quad_body(i, carry):
        for u in range(NB):
          step(NB * i + u, u, u == 0)
        return carry
      lax.fori_loop(0, nch // NB, quad_body, 0)
      for t in range(nch % NB):
        step((nch // NB) * NB + t, t, False)
      scatter_wait(nch - 1, bufs[(nch - 1) % NB])

      plsc.subcore_barrier()
      for t in range(-(-NBLK // NS)):
        b = s + NS * t
        @pl.when(b < NBLK)
        def _readout():
          pltpu.sync_copy(acc.at[pl.ds(b * BLK, BLK)],
                          out.at[c, k, pl.ds(b * BLK, BLK)])

  return pl.kernel(
      body,
      out_type=jax.ShapeDtypeStruct((NC, ET, NN, 16), F32),
      mesh=mesh,
      scratch_types=scratch,
      compiler_params=pltpu.CompilerParams(needs_layout_passes=False,
                                           use_tc_tiling_on_sc=False),
  )


_EDGE_K1 = _make_edge_kernel(ETS_C1)
_EDGE_K2 = _make_edge_kernel(ETS_C2)

_full = lambda shp: pl.BlockSpec(shp, lambda *_: tuple(0 for _ in shp))
_rows = lambda shp: pl.BlockSpec(shp, lambda *gi: (gi[-1],) + (0,) * (len(shp) - 1))

_REC_SDS = jax.ShapeDtypeStruct((NN, 16), F32)
_AL_SDS = jax.ShapeDtypeStruct((NN, 1), F32)
_TBL_OUT = ([_REC_SDS] * 3) + ([_AL_SDS] * 8)
_TBL_OUT_SPECS = ([_rows((BM, 16))] * 3) + ([_rows((BM, 1))] * 8)


def kernel(x_ind, x_org, x_ext, edge_index_txn, edge_index_rev_txn,
           edge_index_link, edge_index_rev_link, edge_attr_txn,
           c1_Wp_ind, c1_Wp_org, c1_Wp_ext, c1_bp_ind, c1_bp_org, c1_bp_ext,
           c1_asrc_ind__txn__org, c1_adst_ind__txn__org,
           c1_asrc_org__rev_txn__ind, c1_adst_org__rev_txn__ind,
           c1_asrc_ext__link__ind, c1_adst_ext__link__ind,
           c1_asrc_ind__rev_link__ext, c1_adst_ind__rev_link__ext,
           c1_kW, c1_kb, c1_q,
           c2_Wp_ind, c2_Wp_org, c2_Wp_ext, c2_bp_ind, c2_bp_org, c2_bp_ext,
           c2_asrc_ind__txn__org, c2_adst_ind__txn__org,
           c2_asrc_org__rev_txn__ind, c2_adst_org__rev_txn__ind,
           c2_asrc_ext__link__ind, c2_adst_ext__link__ind,
           c2_asrc_ind__rev_link__ext, c2_adst_ind__rev_link__ext,
           c2_kW, c2_kb, c2_q,
           lin_ind_1_W, lin_ind_1_b, lin_org_1_W, lin_org_1_b,
           lin_ext_1_W, lin_ext_1_b,
           lin_ind_2_W, lin_ind_2_b, lin_org_2_W, lin_org_2_b):
  col = lambda a: a.reshape(8, 1)

  t1 = pl.pallas_call(
      _pre1_body,
      grid=(GG,),
      in_specs=[_rows((BM, 128))] * 3 + [_full((128, 8))] * 3
      + [_full((8,))] * 3 + [_full((8, 1))] * 8,
      out_specs=_TBL_OUT_SPECS,
      out_shape=_TBL_OUT,
  )(x_ind, x_org, x_ext, c1_Wp_ind, c1_Wp_org, c1_Wp_ext,
    c1_bp_ind, c1_bp_org, c1_bp_ext,
    col(c1_asrc_ind__txn__org), col(c1_adst_ind__txn__org),
    col(c1_asrc_org__rev_txn__ind), col(c1_adst_org__rev_txn__ind),
    col(c1_asrc_ext__link__ind), col(c1_adst_ext__link__ind),
    col(c1_asrc_ind__rev_link__ext), col(c1_adst_ind__rev_link__ext))
  recs1, alphas1 = t1[0:3], t1[3:11]

  eis = (edge_index_txn, edge_index_rev_txn, edge_index_link,
         edge_index_rev_link)

  zeros16 = jnp.zeros((BLK, 16), F32)

  def edge_inputs(ets, recs, alphas):
    args = []
    for ei, (snt, dnt, ne, rs, rd) in zip(eis, ets):
      args += [ei.reshape(2, ne // CHUNK, CHUNK),
               alphas[ROLE_IDX[(snt, rs)]].reshape(NN),
               alphas[ROLE_IDX[(dnt, rd)]].reshape(NN)]
    return args + list(recs) + [zeros16]

  acc1 = _EDGE_K1(*edge_inputs(ETS_C1, recs1, alphas1))

  t2 = pl.pallas_call(
      _mid_body,
      grid=(2, GG),
      in_specs=[pl.BlockSpec((2, 4, BM, 16), lambda p, g: (0, 0, g, 0))]
      + [_full((8, 8)), _full((8,)), _full((8,))]
      + [_full((8, 128)), _full((128,))] * 3
      + [_full((128, 8))] * 3 + [_full((8,))] * 3 + [_full((8, 1))] * 6,
      out_specs=_TBL_OUT_SPECS,
      out_shape=_TBL_OUT,
      scratch_shapes=[pltpu.SMEM((2,), F32)],
  )(acc1, c1_kW, c1_kb, c1_q,
    lin_ind_1_W, lin_ind_1_b, lin_org_1_W, lin_org_1_b,
    lin_ext_1_W, lin_ext_1_b,
    c2_Wp_ind, c2_Wp_org, c2_Wp_ext, c2_bp_ind, c2_bp_org, c2_bp_ext,
    col(c2_asrc_ind__txn__org), col(c2_adst_ind__txn__org),
    col(c2_asrc_org__rev_txn__ind), col(c2_adst_org__rev_txn__ind),
    col(c2_asrc_ext__link__ind), col(c2_adst_ext__link__ind))
  recs2, alphas2 = t2[0:3], t2[3:11]

  acc2 = _EDGE_K2(*edge_inputs(ETS_C2, recs2, alphas2))

  zi, zo = pl.pallas_call(
      _fin_body,
      grid=(2, GG),
      in_specs=[pl.BlockSpec((2, 3, BM, 16), lambda p, g: (0, 0, g, 0))]
      + [_full((8, 8)), _full((8,)), _full((8,))]
      + [_full((8, 1)), _full((1,))] * 2,
      out_specs=[_rows((BM, 1))] * 2,
      out_shape=[jax.ShapeDtypeStruct((NN, 1), F32),
                 jax.ShapeDtypeStruct((NN, 1), F32)],
      scratch_shapes=[pltpu.SMEM((2,), F32)],
  )(acc2, c2_kW, c2_kb, c2_q,
    lin_ind_2_W, lin_ind_2_b, lin_org_2_W, lin_org_2_b)
  return (zi[:, 0], zo[:, 0])


# speedup vs baseline: 1.8224x; 1.0442x over previous
import functools
import numpy as np
import jax
import jax.numpy as jnp
from jax import lax
from jax.experimental import pallas as pl
from jax.experimental.pallas import tpu as pltpu
from jax.experimental.pallas import tpu_sc as plsc

NN = 20000
F32 = jnp.float32
NC = 2
NS = 16
NW = NC * NS
CHUNK = 80
BLK = 1000
NBLK = NN // BLK
BM = 2000
GG = NN // BM

NTS = ("ind", "org", "ext")
ROLES = [("ind", 0), ("ind", 1), ("ind", 2), ("ind", 3),
         ("org", 0), ("org", 1), ("ext", 0), ("ext", 1)]
ROLE_IDX = {r: i for i, r in enumerate(ROLES)}
NROLE = {"ind": 4, "org": 2, "ext": 2}
ETS_C1 = [
    ("ind", "org", 640000, 0, 1),
    ("org", "ind", 640000, 0, 2),
    ("ext", "ind", 320000, 0, 3),
    ("ind", "ext", 320000, 1, 1),
]
ETS_C2 = ETS_C1[:3]


def _emit_block(src, wps, bps, avecs, rec_refs, alpha_refs):
  for i, nt in enumerate(NTS):
    xn = jnp.dot(src[nt], wps[i][:], preferred_element_type=F32) + bps[i][:]
    rec_refs[i][:, :] = jnp.concatenate(
        [xn, jnp.ones((BM, 1), F32), jnp.zeros((BM, 7), F32)], axis=1)
    for r in range(NROLE[nt]):
      av = avecs.get((nt, r))
      alpha_refs[ROLE_IDX[(nt, r)]][:, :] = (
          jnp.zeros((BM, 1), F32) if av is None
          else jnp.dot(xn, av[:], preferred_element_type=F32))


def _pre1_body(x_i, x_o, x_e, wp_i, wp_o, wp_e, bp_i, bp_o, bp_e,
               as_txn, ad_txn, as_rt, ad_rt, as_l, ad_l, as_rl, ad_rl,
               *outs):
  src = {"ind": x_i[:], "org": x_o[:], "ext": x_e[:]}
  avecs = {("ind", 0): as_txn, ("ind", 1): as_rl, ("ind", 2): ad_rt,
           ("ind", 3): ad_l, ("org", 0): as_rt, ("org", 1): ad_txn,
           ("ext", 0): as_l, ("ext", 1): ad_rl}
  _emit_block(src, (wp_i, wp_o, wp_e), (bp_i, bp_o, bp_e), avecs,
              outs[0:3], outs[3:11])


def _seg_out(acc_ref, k):
  a0 = acc_ref[0, k]
  a1 = acc_ref[1, k]
  num = a0[:, 0:8] + a1[:, 0:8]
  den = a0[:, 8:9] + a1[:, 8:9]
  return jnp.maximum(num / (den + 1e-16), 0.0)


def _score_part(o, kW, kb, q):
  t = jnp.tanh(jnp.dot(o, kW, preferred_element_type=F32) + kb)
  return jnp.sum(t * q)


def _attn2(sm):
  m1 = sm[0] / NN
  m2 = sm[1] / NN
  mm = jnp.maximum(m1, m2)
  e1 = jnp.exp(m1 - mm)
  e2 = jnp.exp(m2 - mm)
  return e1 / (e1 + e2), e2 / (e1 + e2)


def _mid_body(acc, kW, kb, q,
              l1iw, l1ib, l1ow, l1ob, l1ew, l1eb,
              wp_i, wp_o, wp_e, bp_i, bp_o, bp_e,
              as_txn, ad_txn, as_rt, ad_rt, as_l, ad_l,
              *outs_and_scratch):
  outs = outs_and_scratch[:-1]
  sm = outs_and_scratch[-1]
  p = pl.program_id(0)
  g = pl.program_id(1)
  o_rt = _seg_out(acc, 1)
  o_l = _seg_out(acc, 2)

  @pl.when(jnp.logical_and(p == 0, g == 0))
  def _init():
    sm[0] = 0.0
    sm[1] = 0.0

  @pl.when(p == 0)
  def _accum():
    sm[0] = sm[0] + _score_part(o_rt, kW[:], kb[:], q[:])
    sm[1] = sm[1] + _score_part(o_l, kW[:], kb[:], q[:])

  @pl.when(p == 1)
  def _emit():
    a1, a2 = _attn2(sm)
    res = {"org": _seg_out(acc, 0),
           "ind": a1 * o_rt + a2 * o_l,
           "ext": _seg_out(acc, 3)}
    lin = {"ind": (l1iw, l1ib), "org": (l1ow, l1ob), "ext": (l1ew, l1eb)}
    src = {nt: jax.nn.sigmoid(
        jnp.dot(res[nt], lin[nt][0][:], preferred_element_type=F32)
        + lin[nt][1][:]) for nt in NTS}
    avecs = {("ind", 0): as_txn, ("ind", 2): ad_rt, ("ind", 3): ad_l,
             ("org", 0): as_rt, ("org", 1): ad_txn, ("ext", 0): as_l}
    _emit_block(src, (wp_i, wp_o, wp_e), (bp_i, bp_o, bp_e), avecs,
                outs[0:3], outs[3:11])


def _fin_body(acc, kW, kb, q, l2iw, l2ib, l2ow, l2ob, zi_ref, zo_ref, sm):
  p = pl.program_id(0)
  g = pl.program_id(1)
  o_rt = _seg_out(acc, 1)
  o_l = _seg_out(acc, 2)

  @pl.when(jnp.logical_and(p == 0, g == 0))
  def _init():
    sm[0] = 0.0
    sm[1] = 0.0

  @pl.when(p == 0)
  def _accum():
    sm[0] = sm[0] + _score_part(o_rt, kW[:], kb[:], q[:])
    sm[1] = sm[1] + _score_part(o_l, kW[:], kb[:], q[:])

  @pl.when(p == 1)
  def _emit():
    a1, a2 = _attn2(sm)
    res_ind = a1 * o_rt + a2 * o_l
    zi_ref[:, :] = jax.nn.sigmoid(
        jnp.dot(res_ind, l2iw[:], preferred_element_type=F32) + l2ib[:])
    zo_ref[:, :] = jax.nn.sigmoid(
        jnp.dot(_seg_out(acc, 0), l2ow[:], preferred_element_type=F32)
        + l2ob[:])


MAXCH = 250


def _make_edge_kernel(ets):
  ET = len(ets)
  mesh = plsc.VectorSubcoreMesh(core_axis_name="c", subcore_axis_name="s")
  NB = 8
  scratch = [
      pltpu.VMEM((NN,), F32),
      pltpu.VMEM((NN,), F32),
      pltpu.VMEM((2, MAXCH, CHUNK), jnp.int32),
      pltpu.VMEM_SHARED((NN, 16), F32),
  ] + [
      pltpu.VMEM((CHUNK, 16), F32), pltpu.VMEM((CHUNK,), F32),
      pltpu.SemaphoreType.DMA, pltpu.SemaphoreType.DMA,
  ] * NB

  def body(*refs):
    eis = [refs[3 * k] for k in range(ET)]
    asrcs = [refs[3 * k + 1] for k in range(ET)]
    adsts = [refs[3 * k + 2] for k in range(ET)]
    rec = {"ind": refs[3 * ET], "org": refs[3 * ET + 1],
           "ext": refs[3 * ET + 2]}
    zeros16 = refs[3 * ET + 3]
    out = refs[3 * ET + 4]
    (asrc_t, adst_t, idx_all, acc) = refs[3 * ET + 5: 3 * ET + 9]
    base = 3 * ET + 9
    bufs = [refs[base + 4 * i: base + 4 * (i + 1)] for i in range(NB)]

    c = lax.axis_index("c")
    s = lax.axis_index("s")
    w = c * NS + s

    def table_max(tbl):
      def mbody(r, m):
        return jnp.maximum(m, tbl[pl.ds(r * 16, 16)])
      m = lax.fori_loop(0, NN // 16, mbody, jnp.full((16,), -3.4e38, F32))
      return jnp.max(m)

    for k, (snt, dnt, ne, rs, rd) in enumerate(ets):
      for t in range(-(-NBLK // NS)):
        b = s + NS * t
        @pl.when(b < NBLK)
        def _zero():
          pltpu.sync_copy(zeros16, acc.at[pl.ds(b * BLK, BLK)])
      pltpu.sync_copy(asrcs[k], asrc_t)
      pltpu.sync_copy(adsts[k], adst_t)
      nch = ne // NW // CHUNK
      pltpu.sync_copy(eis[k].at[:, pl.ds(w * nch, nch)],
                      idx_all.at[:, pl.ds(0, nch)])
      sv = jnp.full(
          (16,),
          jnp.maximum(table_max(asrc_t) + table_max(adst_t), 0.0), F32)
      rec_hbm = rec[snt]
      plsc.subcore_barrier()

      def gather(ci, buf):
        rows, exb, gsem, ssem = buf
        pltpu.async_copy(rec_hbm.at[idx_all.at[0, ci]], rows, gsem)

      def gather_wait(ci, buf):
        rows, exb, gsem, ssem = buf
        pltpu.make_async_copy(rec_hbm.at[idx_all.at[0, ci]], rows,
                              gsem).wait()

      def scatter(ci, buf):
        rows, exb, gsem, ssem = buf
        pltpu.async_copy(rows, acc.at[idx_all.at[1, ci]], ssem, add=True)

      def scatter_wait(ci, buf):
        rows, exb, gsem, ssem = buf
        pltpu.make_async_copy(rows, acc.at[idx_all.at[1, ci]], ssem).wait()

      def alpha_pass(ci, buf, sv=sv):
        rows, exb, gsem, ssem = buf
        for j in range(CHUNK // 16):
          sl = pl.ds(j * 16, 16)
          a = (plsc.load_gather(asrc_t, [idx_all[0, ci, sl]])
               + plsc.load_gather(adst_t, [idx_all[1, ci, sl]]))
          a = jnp.where(a >= 0.0, a, a * 0.2)
          exb[sl] = jnp.exp(a - sv)

      def escale(buf):
        rows, exb, gsem, ssem = buf

        @plsc.parallel_loop(0, CHUNK, 1, unroll=8)
        def _loop(e):
          bex = plsc.load_gather(exb, [jnp.full((16,), e, jnp.int32)])
          rows[e, :] = rows[e, :] * bex

      for u in range(NB - 1):
        gather(u, bufs[u])

      def step(ci, u, first):
        alpha_pass(ci, bufs[u])
        gather_wait(ci, bufs[u])
        pv = (u + NB - 1) % NB
        if first:
          @pl.when(ci >= 1)
          def _drain():
            scatter_wait(ci - 1, bufs[pv])
        else:
          scatter_wait(ci - 1, bufs[pv])

        @pl.when(ci + NB - 1 < nch)
        def _pref():
          gather(ci + NB - 1, bufs[pv])
        escale(bufs[u])
        scatter(ci, bufs[u])

      def quad_body(i, carry):
        for u in range(NB):
          step(NB * i + u, u, u == 0)
        return carry
      lax.fori_loop(0, nch // NB, quad_body, 0)
      for t in range(nch % NB):
        step((nch // NB) * NB + t, t, False)
      scatter_wait(nch - 1, bufs[(nch - 1) % NB])

      plsc.subcore_barrier()
      for t in range(-(-NBLK // NS)):
        b = s + NS * t
        @pl.when(b < NBLK)
        def _readout():
          pltpu.sync_copy(acc.at[pl.ds(b * BLK, BLK)],
                          out.at[c, k, pl.ds(b * BLK, BLK)])

  return pl.kernel(
      body,
      out_type=jax.ShapeDtypeStruct((NC, ET, NN, 16), F32),
      mesh=mesh,
      scratch_types=scratch,
      compiler_params=pltpu.CompilerParams(needs_layout_passes=False,
                                           use_tc_tiling_on_sc=False),
  )


_EDGE_K1 = _make_edge_kernel(ETS_C1)
_EDGE_K2 = _make_edge_kernel(ETS_C2)

_full = lambda shp: pl.BlockSpec(shp, lambda *_: tuple(0 for _ in shp))
_rows = lambda shp: pl.BlockSpec(shp, lambda *gi: (gi[-1],) + (0,) * (len(shp) - 1))

_REC_SDS = jax.ShapeDtypeStruct((NN, 16), F32)
_AL_SDS = jax.ShapeDtypeStruct((NN, 1), F32)
_TBL_OUT = ([_REC_SDS] * 3) + ([_AL_SDS] * 8)
_TBL_OUT_SPECS = ([_rows((BM, 16))] * 3) + ([_rows((BM, 1))] * 8)


def kernel(x_ind, x_org, x_ext, edge_index_txn, edge_index_rev_txn,
           edge_index_link, edge_index_rev_link, edge_attr_txn,
           c1_Wp_ind, c1_Wp_org, c1_Wp_ext, c1_bp_ind, c1_bp_org, c1_bp_ext,
           c1_asrc_ind__txn__org, c1_adst_ind__txn__org,
           c1_asrc_org__rev_txn__ind, c1_adst_org__rev_txn__ind,
           c1_asrc_ext__link__ind, c1_adst_ext__link__ind,
           c1_asrc_ind__rev_link__ext, c1_adst_ind__rev_link__ext,
           c1_kW, c1_kb, c1_q,
           c2_Wp_ind, c2_Wp_org, c2_Wp_ext, c2_bp_ind, c2_bp_org, c2_bp_ext,
           c2_asrc_ind__txn__org, c2_adst_ind__txn__org,
           c2_asrc_org__rev_txn__ind, c2_adst_org__rev_txn__ind,
           c2_asrc_ext__link__ind, c2_adst_ext__link__ind,
           c2_asrc_ind__rev_link__ext, c2_adst_ind__rev_link__ext,
           c2_kW, c2_kb, c2_q,
           lin_ind_1_W, lin_ind_1_b, lin_org_1_W, lin_org_1_b,
           lin_ext_1_W, lin_ext_1_b,
           lin_ind_2_W, lin_ind_2_b, lin_org_2_W, lin_org_2_b):
  col = lambda a: a.reshape(8, 1)

  t1 = pl.pallas_call(
      _pre1_body,
      grid=(GG,),
      in_specs=[_rows((BM, 128))] * 3 + [_full((128, 8))] * 3
      + [_full((8,))] * 3 + [_full((8, 1))] * 8,
      out_specs=_TBL_OUT_SPECS,
      out_shape=_TBL_OUT,
  )(x_ind, x_org, x_ext, c1_Wp_ind, c1_Wp_org, c1_Wp_ext,
    c1_bp_ind, c1_bp_org, c1_bp_ext,
    col(c1_asrc_ind__txn__org), col(c1_adst_ind__txn__org),
    col(c1_asrc_org__rev_txn__ind), col(c1_adst_org__rev_txn__ind),
    col(c1_asrc_ext__link__ind), col(c1_adst_ext__link__ind),
    col(c1_asrc_ind__rev_link__ext), col(c1_adst_ind__rev_link__ext))
  recs1, alphas1 = t1[0:3], t1[3:11]

  eis = (edge_index_txn, edge_index_rev_txn, edge_index_link,
         edge_index_rev_link)

  zeros16 = jnp.zeros((BLK, 16), F32)

  def edge_inputs(ets, recs, alphas):
    args = []
    for ei, (snt, dnt, ne, rs, rd) in zip(eis, ets):
      args += [ei.reshape(2, ne // CHUNK, CHUNK),
               alphas[ROLE_IDX[(snt, rs)]].reshape(NN),
               alphas[ROLE_IDX[(dnt, rd)]].reshape(NN)]
    return args + list(recs) + [zeros16]

  acc1 = _EDGE_K1(*edge_inputs(ETS_C1, recs1, alphas1))

  t2 = pl.pallas_call(
      _mid_body,
      grid=(2, GG),
      in_specs=[pl.BlockSpec((2, 4, BM, 16), lambda p, g: (0, 0, g, 0))]
      + [_full((8, 8)), _full((8,)), _full((8,))]
      + [_full((8, 128)), _full((128,))] * 3
      + [_full((128, 8))] * 3 + [_full((8,))] * 3 + [_full((8, 1))] * 6,
      out_specs=_TBL_OUT_SPECS,
      out_shape=_TBL_OUT,
      scratch_shapes=[pltpu.SMEM((2,), F32)],
  )(acc1, c1_kW, c1_kb, c1_q,
    lin_ind_1_W, lin_ind_1_b, lin_org_1_W, lin_org_1_b,
    lin_ext_1_W, lin_ext_1_b,
    c2_Wp_ind, c2_Wp_org, c2_Wp_ext, c2_bp_ind, c2_bp_org, c2_bp_ext,
    col(c2_asrc_ind__txn__org), col(c2_adst_ind__txn__org),
    col(c2_asrc_org__rev_txn__ind), col(c2_adst_org__rev_txn__ind),
    col(c2_asrc_ext__link__ind), col(c2_adst_ext__link__ind))
  recs2, alphas2 = t2[0:3], t2[3:11]

  acc2 = _EDGE_K2(*edge_inputs(ETS_C2, recs2, alphas2))

  zi, zo = pl.pallas_call(
      _fin_body,
      grid=(2, GG),
      in_specs=[pl.BlockSpec((2, 3, BM, 16), lambda p, g: (0, 0, g, 0))]
      + [_full((8, 8)), _full((8,)), _full((8,))]
      + [_full((8, 1)), _full((1,))] * 2,
      out_specs=[_rows((BM, 1))] * 2,
      out_shape=[jax.ShapeDtypeStruct((NN, 1), F32),
                 jax.ShapeDtypeStruct((NN, 1), F32)],
      scratch_shapes=[pltpu.SMEM((2,), F32)],
  )(acc2, c2_kW, c2_kb, c2_q,
    lin_ind_2_W, lin_ind_2_b, lin_org_2_W, lin_org_2_b)
  return (zi[:, 0], zo[:, 0])
